# baseline (speedup 1.0000x reference)
import functools

import jax
import jax.numpy as jnp
from jax import lax
from jax.experimental import pallas as pl
from jax.experimental.pallas import tpu as pltpu
from jax.experimental.pallas import tpu_sc as plsc

_NUM_CORES = 2
_NUM_SUBCORES = 16
_NW = _NUM_CORES * _NUM_SUBCORES
_IDXW = 128
_K = 2
_BS = 1024


@functools.cache
def _gather_build(B, D, ci):
    R = B // _K
    rpw = R // _NW
    nsub = rpw // _IDXW
    mesh = plsc.VectorSubcoreMesh(core_axis_name="c", subcore_axis_name="s")

    @functools.partial(
        pl.kernel,
        out_type=jax.ShapeDtypeStruct((R, D), jnp.float32),
        mesh=mesh,
        scratch_types=[
            pltpu.VMEM((nsub, _IDXW), jnp.int32),
            pltpu.VMEM((nsub, _IDXW, D), jnp.float32),
            [pltpu.SemaphoreType.DMA] * nsub,
            pltpu.SemaphoreType.DMA,
        ],
    )
    def gk(label_hbm, centers_hbm, out_hbm, idx_v, gbuf_v, gsems, wsem):
        wid = lax.axis_index("s") * _NUM_CORES + lax.axis_index("c")
        blk = ci * (R // _IDXW) + wid * nsub
        pltpu.sync_copy(label_hbm.at[pl.ds(blk, nsub), :], idx_v)
        gds = [
            pltpu.async_copy(centers_hbm.at[idx_v.at[s]],
                             gbuf_v.at[s], gsems[s])
            for s in range(nsub)
        ]
        wds = []
        for s in range(nsub):
            gds[s].wait()
            wds.append(pltpu.async_copy(
                gbuf_v.at[s],
                out_hbm.at[pl.ds(wid * rpw + s * _IDXW, _IDXW), :],
                wsem))
        for w in wds:
            w.wait()

    return gk


@functools.cache
def _reduce_build(B, D, ci):
    R = B // _K
    grid = R // _BS
    base = ci * R // _BS

    def rk(feat_ref, a_ref, g_ref, o_ref):
        d = feat_ref[...] - g_ref[...]
        t = a_ref[...] * d * d
        o_ref[...] = jnp.broadcast_to(
            jnp.sum(t, axis=0, keepdims=True), (8, D))

    return pl.pallas_call(
        rk,
        grid=(grid,),
        in_specs=[
            pl.BlockSpec((_BS, D), lambda i: (base + i, 0)),
            pl.BlockSpec((_BS, D), lambda i: (base + i, 0)),
            pl.BlockSpec((_BS, D), lambda i: (i, 0)),
        ],
        out_specs=pl.BlockSpec((8, D), lambda i: (i, 0)),
        out_shape=jax.ShapeDtypeStruct((grid * 8, D), jnp.float32),
    )


def kernel(feat, A, label, centers):
    B, D = feat.shape
    label2d = label.astype(jnp.int32).reshape(B // _IDXW, _IDXW)
    parts = []
    for ci in range(_K):
        g = _gather_build(B, D, ci)(label2d, centers)
        parts.append(_reduce_build(B, D, ci)(feat, A, g))
    total = parts[0]
    for p in parts[1:]:
        total = total + p
    return jnp.sum(total) * (0.5 / B / 8)


# speedup vs baseline: 1.0696x; 1.0696x over previous
import functools

import jax
import jax.numpy as jnp
from jax import lax
from jax.experimental import pallas as pl
from jax.experimental.pallas import tpu as pltpu
from jax.experimental.pallas import tpu_sc as plsc

_NUM_CORES = 2
_NUM_SUBCORES = 16
_NW = _NUM_CORES * _NUM_SUBCORES
_IDXW = 128
_K = 2
_BS = 2048


@functools.cache
def _gather_build(B, D, ci):
    R = B // _K
    rpw = R // _NW
    nsub = rpw // _IDXW
    mesh = plsc.VectorSubcoreMesh(core_axis_name="c", subcore_axis_name="s")

    @functools.partial(
        pl.kernel,
        out_type=jax.ShapeDtypeStruct((R, D), jnp.float32),
        mesh=mesh,
        scratch_types=[
            pltpu.VMEM((nsub, _IDXW), jnp.int32),
            pltpu.VMEM((nsub, _IDXW, D), jnp.float32),
            [pltpu.SemaphoreType.DMA] * nsub,
            pltpu.SemaphoreType.DMA,
        ],
    )
    def gk(label_hbm, centers_hbm, out_hbm, idx_v, gbuf_v, gsems, wsem):
        wid = lax.axis_index("s") * _NUM_CORES + lax.axis_index("c")
        blk = ci * (R // _IDXW) + wid * nsub
        pltpu.sync_copy(label_hbm.at[pl.ds(blk, nsub), :], idx_v)
        gds = [
            pltpu.async_copy(centers_hbm.at[idx_v.at[s]],
                             gbuf_v.at[s], gsems[s])
            for s in range(nsub)
        ]
        wds = []
        for s in range(nsub):
            gds[s].wait()
            wds.append(pltpu.async_copy(
                gbuf_v.at[s],
                out_hbm.at[pl.ds(wid * rpw + s * _IDXW, _IDXW), :],
                wsem))
        for w in wds:
            w.wait()

    return gk


@functools.cache
def _reduce_build(B, D, ci):
    R = B // _K
    grid = R // _BS
    base = ci * R // _BS

    def rk(feat_ref, a_ref, g_ref, o_ref):
        d = feat_ref[...] - g_ref[...]
        t = a_ref[...] * d * d
        o_ref[...] = jnp.broadcast_to(
            jnp.sum(t, axis=0, keepdims=True), (8, D))

    return pl.pallas_call(
        rk,
        grid=(grid,),
        in_specs=[
            pl.BlockSpec((_BS, D), lambda i: (base + i, 0)),
            pl.BlockSpec((_BS, D), lambda i: (base + i, 0)),
            pl.BlockSpec((_BS, D), lambda i: (i, 0)),
        ],
        out_specs=pl.BlockSpec((8, D), lambda i: (i, 0)),
        out_shape=jax.ShapeDtypeStruct((grid * 8, D), jnp.float32),
    )


def kernel(feat, A, label, centers):
    B, D = feat.shape
    label2d = label.astype(jnp.int32).reshape(B // _IDXW, _IDXW)
    parts = []
    for ci in range(_K):
        g = _gather_build(B, D, ci)(label2d, centers)
        parts.append(_reduce_build(B, D, ci)(feat, A, g))
    total = parts[0]
    for p in parts[1:]:
        total = total + p
    return jnp.sum(total) * (0.5 / B / 8)


# speedup vs baseline: 1.1247x; 1.0515x over previous
import functools

import jax
import jax.numpy as jnp
from jax import lax
from jax.experimental import pallas as pl
from jax.experimental.pallas import tpu as pltpu
from jax.experimental.pallas import tpu_sc as plsc

_NUM_CORES = 2
_NUM_SUBCORES = 16
_NW = _NUM_CORES * _NUM_SUBCORES
_LANES = 16
_CHUNK = 128
_BS = 2048


@functools.cache
def _sc_build(B, D):
    rows_per_w = B // _NW
    n_chunks = rows_per_w // _CHUNK
    vecs_per_row = D // _LANES
    mesh = plsc.VectorSubcoreMesh(core_axis_name="c", subcore_axis_name="s")

    @functools.partial(
        pl.kernel,
        out_type=jax.ShapeDtypeStruct((_NW * _LANES,), jnp.float32),
        mesh=mesh,
        scratch_types=[
            pltpu.VMEM((n_chunks, _CHUNK), jnp.int32),
            pltpu.VMEM((2, _CHUNK, D), jnp.float32),
            pltpu.VMEM((2, _CHUNK, D), jnp.float32),
            pltpu.VMEM((2, _CHUNK, D), jnp.float32),
            pltpu.VMEM((_LANES,), jnp.float32),
            [pltpu.SemaphoreType.DMA] * 6,
        ],
    )
    def sc_kernel(feat_hbm, a_hbm, label_hbm, centers_hbm, out_hbm,
                  idx_v, cent_v, feat_v, a_v, acc_v, sems):
        wid = lax.axis_index("s") * _NUM_CORES + lax.axis_index("c")
        base = wid * rows_per_w
        pltpu.sync_copy(label_hbm.at[pl.ds(wid * n_chunks, n_chunks), :],
                        idx_v)

        def fire(ci, slot):
            row0 = base + ci * _CHUNK
            return (
                pltpu.async_copy(centers_hbm.at[idx_v.at[ci]],
                                 cent_v.at[slot], sems[3 * slot]),
                pltpu.async_copy(feat_hbm.at[pl.ds(row0, _CHUNK), :],
                                 feat_v.at[slot], sems[3 * slot + 1]),
                pltpu.async_copy(a_hbm.at[pl.ds(row0, _CHUNK), :],
                                 a_v.at[slot], sems[3 * slot + 2]),
            )

        acc = tuple(jnp.zeros((_LANES,), jnp.float32)
                    for _ in range(vecs_per_row))
        in_flight = fire(0, 0)
        for ci in range(n_chunks):
            slot = ci % 2
            cur = in_flight
            if ci + 1 < n_chunks:
                in_flight = fire(ci + 1, 1 - slot)
            for cp in cur:
                cp.wait()

            def row_body(r, accs):
                new = []
                for j in range(vecs_per_row):
                    f = feat_v[slot, r, pl.ds(j * _LANES, _LANES)]
                    c = cent_v[slot, r, pl.ds(j * _LANES, _LANES)]
                    w = a_v[slot, r, pl.ds(j * _LANES, _LANES)]
                    new.append(accs[j] + (w * c) * (c - (f + f)))
                return tuple(new)

            acc = lax.fori_loop(0, _CHUNK, row_body, acc)
        total = acc[0]
        for j in range(1, vecs_per_row):
            total = total + acc[j]
        acc_v[...] = total
        pltpu.sync_copy(acc_v, out_hbm.at[pl.ds(wid * _LANES, _LANES)])

    return sc_kernel


@functools.cache
def _tc_build(B, D):
    grid = B // _BS

    def rk(feat_ref, a_ref, o_ref):
        f = feat_ref[...]
        t = a_ref[...] * f * f
        o_ref[...] = jnp.broadcast_to(
            jnp.sum(t, axis=0, keepdims=True), (8, D))

    return pl.pallas_call(
        rk,
        grid=(grid,),
        in_specs=[
            pl.BlockSpec((_BS, D), lambda i: (i, 0)),
            pl.BlockSpec((_BS, D), lambda i: (i, 0)),
        ],
        out_specs=pl.BlockSpec((8, D), lambda i: (i, 0)),
        out_shape=jax.ShapeDtypeStruct((grid * 8, D), jnp.float32),
    )


def kernel(feat, A, label, centers):
    B, D = feat.shape
    label2d = label.astype(jnp.int32).reshape(B // _CHUNK, _CHUNK)
    sc_part = _sc_build(B, D)(feat, A, label2d, centers)
    tc_part = _tc_build(B, D)(feat, A)
    total = jnp.sum(sc_part) + jnp.sum(tc_part) * (1.0 / 8.0)
    return total * (0.5 / B)


# speedup vs baseline: 1.1786x; 1.0479x over previous
import functools

import jax
import jax.numpy as jnp
from jax import lax
from jax.experimental import pallas as pl
from jax.experimental.pallas import tpu as pltpu
from jax.experimental.pallas import tpu_sc as plsc

_NUM_CORES = 2
_NUM_SUBCORES = 16
_NW = _NUM_CORES * _NUM_SUBCORES
_LANES = 16
_CHUNK = 64


@functools.cache
def _build(B, D):
    rows_per_w = B // _NW
    n_chunks = rows_per_w // _CHUNK
    vecs_per_row = D // _LANES
    assert rows_per_w * _NW == B and n_chunks * _CHUNK == rows_per_w
    assert vecs_per_row * _LANES == D

    mesh = plsc.VectorSubcoreMesh(core_axis_name="c", subcore_axis_name="s")

    @functools.partial(
        pl.kernel,
        out_type=jax.ShapeDtypeStruct((_NW * _LANES,), jnp.float32),
        mesh=mesh,
        scratch_types=[
            pltpu.VMEM((n_chunks, _CHUNK), jnp.int32),
            pltpu.VMEM((2, _CHUNK, D), jnp.float32),
            pltpu.VMEM((2, _CHUNK, D), jnp.float32),
            pltpu.VMEM((2, _CHUNK, D), jnp.float32),
            pltpu.VMEM((_LANES,), jnp.float32),
            [pltpu.SemaphoreType.DMA] * 6,
        ],
    )
    def sc_kernel(feat_hbm, a_hbm, label_hbm, centers_hbm, out_hbm,
                  idx_v, cent_v, feat_v, a_v, acc_v, sems):
        wid = lax.axis_index("s") * _NUM_CORES + lax.axis_index("c")
        base = wid * rows_per_w

        def fire_linear(ci, slot):
            row0 = base + ci * _CHUNK
            return (
                pltpu.async_copy(feat_hbm.at[pl.ds(row0, _CHUNK), :],
                                 feat_v.at[slot], sems[3 * slot + 1]),
                pltpu.async_copy(a_hbm.at[pl.ds(row0, _CHUNK), :],
                                 a_v.at[slot], sems[3 * slot + 2]),
            )

        def fire_gather(ci, slot):
            return pltpu.async_copy(centers_hbm.at[idx_v.at[ci]],
                                    cent_v.at[slot], sems[3 * slot])

        lin0 = fire_linear(0, 0)
        pltpu.sync_copy(label_hbm.at[pl.ds(wid * n_chunks, n_chunks), :],
                        idx_v)
        in_flight = lin0 + (fire_gather(0, 0),)

        acc = tuple(jnp.zeros((_LANES,), jnp.float32)
                    for _ in range(vecs_per_row))
        for ci in range(n_chunks):
            slot = ci % 2
            cur = in_flight
            if ci + 1 < n_chunks:
                in_flight = ((fire_gather(ci + 1, 1 - slot),)
                             + fire_linear(ci + 1, 1 - slot))
            for cp in cur:
                cp.wait()

            def row_body(r, accs):
                new = []
                for j in range(vecs_per_row):
                    f = feat_v[slot, r, pl.ds(j * _LANES, _LANES)]
                    c = cent_v[slot, r, pl.ds(j * _LANES, _LANES)]
                    w = a_v[slot, r, pl.ds(j * _LANES, _LANES)]
                    d = f - c
                    new.append(accs[j] + w * d * d)
                return tuple(new)

            acc = lax.fori_loop(0, _CHUNK, row_body, acc)
        total = acc[0]
        for j in range(1, vecs_per_row):
            total = total + acc[j]
        acc_v[...] = total
        pltpu.sync_copy(acc_v, out_hbm.at[pl.ds(wid * _LANES, _LANES)])

    return sc_kernel


def kernel(feat, A, label, centers):
    B, D = feat.shape
    label2d = label.astype(jnp.int32).reshape(B // _CHUNK, _CHUNK)
    partials = _build(B, D)(feat, A, label2d, centers)
    return jnp.sum(partials) * (0.5 / B)


# speedup vs baseline: 1.2073x; 1.0244x over previous
import functools

import jax
import jax.numpy as jnp
from jax import lax
from jax.experimental import pallas as pl
from jax.experimental.pallas import tpu as pltpu
from jax.experimental.pallas import tpu_sc as plsc

_NUM_CORES = 2
_NUM_SUBCORES = 16
_NW = _NUM_CORES * _NUM_SUBCORES
_LANES = 16
_CHUNK = 128


@functools.cache
def _build(B, D):
    rows_per_w = B // _NW
    n_chunks = rows_per_w // _CHUNK
    vecs_per_row = D // _LANES
    assert rows_per_w * _NW == B and n_chunks * _CHUNK == rows_per_w
    assert vecs_per_row * _LANES == D

    mesh = plsc.VectorSubcoreMesh(core_axis_name="c", subcore_axis_name="s")

    @functools.partial(
        pl.kernel,
        out_type=jax.ShapeDtypeStruct((_NW * _LANES,), jnp.float32),
        mesh=mesh,
        scratch_types=[
            pltpu.VMEM((n_chunks, _CHUNK), jnp.int32),
            pltpu.VMEM((2, _CHUNK, D), jnp.float32),
            pltpu.VMEM((2, _CHUNK, D), jnp.float32),
            pltpu.VMEM((2, _CHUNK, D), jnp.float32),
            pltpu.VMEM((_LANES,), jnp.float32),
            [pltpu.SemaphoreType.DMA] * 6,
        ],
    )
    def sc_kernel(feat_hbm, a_hbm, label_hbm, centers_hbm, out_hbm,
                  idx_v, cent_v, feat_v, a_v, acc_v, sems):
        wid = lax.axis_index("s") * _NUM_CORES + lax.axis_index("c")
        base = wid * rows_per_w

        def fire_linear(ci, slot):
            row0 = base + ci * _CHUNK
            return (
                pltpu.async_copy(feat_hbm.at[pl.ds(row0, _CHUNK), :],
                                 feat_v.at[slot], sems[3 * slot + 1]),
                pltpu.async_copy(a_hbm.at[pl.ds(row0, _CHUNK), :],
                                 a_v.at[slot], sems[3 * slot + 2]),
            )

        def fire_gather(ci, slot):
            return pltpu.async_copy(centers_hbm.at[idx_v.at[ci]],
                                    cent_v.at[slot], sems[3 * slot])

        lin0 = fire_linear(0, 0)
        pltpu.sync_copy(label_hbm.at[pl.ds(wid * n_chunks, n_chunks), :],
                        idx_v)
        in_flight = lin0 + (fire_gather(0, 0),)

        acc = tuple(jnp.zeros((_LANES,), jnp.float32)
                    for _ in range(vecs_per_row))
        for ci in range(n_chunks):
            slot = ci % 2
            cur = in_flight
            if ci + 1 < n_chunks:
                in_flight = ((fire_gather(ci + 1, 1 - slot),)
                             + fire_linear(ci + 1, 1 - slot))
            for cp in cur:
                cp.wait()

            def row_body(r, accs):
                new = []
                for j in range(vecs_per_row):
                    f = feat_v[slot, r, pl.ds(j * _LANES, _LANES)]
                    c = cent_v[slot, r, pl.ds(j * _LANES, _LANES)]
                    w = a_v[slot, r, pl.ds(j * _LANES, _LANES)]
                    d = f - c
                    new.append(accs[j] + w * d * d)
                return tuple(new)

            acc = lax.fori_loop(0, _CHUNK, row_body, acc)
        total = acc[0]
        for j in range(1, vecs_per_row):
            total = total + acc[j]
        acc_v[...] = total
        pltpu.sync_copy(acc_v, out_hbm.at[pl.ds(wid * _LANES, _LANES)])

    return sc_kernel


def kernel(feat, A, label, centers):
    B, D = feat.shape
    label2d = label.astype(jnp.int32).reshape(B // _CHUNK, _CHUNK)
    partials = _build(B, D)(feat, A, label2d, centers)
    return jnp.sum(partials) * (0.5 / B)
